# baseline (speedup 1.0000x reference)
import functools

import jax
import jax.numpy as jnp
import numpy as np
from jax.experimental import pallas as pl
from jax.experimental.pallas import tpu as pltpu

_BN_EPS = 1e-5
_C = 128
_G = 4


def _ceil8(v):
    return (v + 7) // 8 * 8


def _geom(W):
    return _ceil8(W + _G + 2)


def _build_shifted(D, t, d, M, Wp, zb):
    dWp = d * Wp
    for kx in range(3):
        b = zb - (kx - 1) * d
        lo, hi = 128 * kx, 128 * (kx + 1)
        above = b - (zb - dWp)
        below = (zb + dWp + M + 8) - (b + M)
        D[pl.ds(zb - dWp, above), lo:hi] = jnp.zeros((above, _C), jnp.float32)
        D[pl.ds(b + M, below), lo:hi] = jnp.zeros((below, _C), jnp.float32)
        D[pl.ds(b, M), lo:hi] = t


def _conv3x3(D, wa, wb, wc, bias, *, Wp, d, M, zb):
    dWp = d * Wp
    Mext = M + dWp
    yA = jnp.dot(D[pl.ds(zb - dWp, Mext), 0:256], wa,
                 preferred_element_type=jnp.float32)
    yB = jnp.dot(D[pl.ds(zb - dWp, Mext), 256:384], wb,
                 preferred_element_type=jnp.float32)
    yC = jnp.dot(D[pl.ds(zb + dWp, M + 8), 0:256], wc,
                 preferred_element_type=jnp.float32)
    acc = (jnp.broadcast_to(bias, (M, _C)).astype(jnp.float32)
           + yA[0:M, 0:128] + yA[dWp:dWp + M, 128:256]
           + yB[0:M, 0:128] + yB[dWp:dWp + M, 128:256]
           + yC[0:M, 0:128] + yC[d:d + M, 128:256])
    return acc


def _stage_kernel(x_ref, col_ref, w00_ref, w0r_ref, b0a_ref,
                  wa_ref, wb_ref, wc_ref, bc_ref,
                  wh1_ref, bh1_ref, wh2_ref, bh2_ref,
                  o_ref, D1, D2, *, Wp, W, M, zb):
    col = col_ref[...]
    valid = (col >= _G) & (col < _G + W)

    x = x_ref[0]
    for b in range(5):
        w0 = w00_ref[...] if b == 0 else w0r_ref[b - 1]
        init = jnp.dot(x, w0, preferred_element_type=jnp.float32)
        init = jnp.where(valid, jnp.maximum(init + b0a_ref[b], 0.0), 0.0)
        _build_shifted(D1, init, 1, M, Wp, zb)
        t = _conv3x3(D1, wa_ref[2 * b], wb_ref[2 * b], wc_ref[2 * b],
                     bc_ref[2 * b], Wp=Wp, d=1, M=M, zb=zb)
        t = jnp.where(valid, jnp.maximum(t, 0.0), 0.0)
        _build_shifted(D2, t, 2, M, Wp, zb)
        t = _conv3x3(D2, wa_ref[2 * b + 1], wb_ref[2 * b + 1], wc_ref[2 * b + 1],
                     bc_ref[2 * b + 1], Wp=Wp, d=2, M=M, zb=zb)
        t = jnp.where(valid, jnp.maximum(t, 0.0), 0.0)
        x = D1[pl.ds(zb, M), 128:256] + t

    m = jnp.dot(x, wh1_ref[...], preferred_element_type=jnp.float32)
    m = jnp.maximum(m + bh1_ref[...], 0.0)
    o_ref[0] = (jnp.dot(m, wh2_ref[...], preferred_element_type=jnp.float32)
                + bh2_ref[...])


def _io(w_oihw):
    return jnp.transpose(w_oihw[:, :, 0, 0], (1, 0))


def _tap_weights_all(tw, tb, tg, tbe, tm, tv):
    scale = tg / jnp.sqrt(tv + _BN_EPS)
    wf = tw * scale[:, :, None, None, None]
    bf = (tb - tm) * scale + tbe
    w = jnp.transpose(wf, (0, 3, 4, 2, 1))
    z = jnp.zeros_like(w[:, 2, 2])
    wa = jnp.concatenate(
        [jnp.concatenate([w[:, 0, 0], w[:, 1, 0]], axis=2),
         jnp.concatenate([w[:, 0, 1], w[:, 1, 1]], axis=2)], axis=1)
    wb = jnp.concatenate([w[:, 0, 2], w[:, 1, 2]], axis=2)
    wc = jnp.concatenate(
        [jnp.concatenate([w[:, 2, 0], z], axis=2),
         jnp.concatenate([w[:, 2, 1], w[:, 2, 2]], axis=2)], axis=1)
    return wa, wb, wc, bf[:, None, :]


def kernel(x, b0_init_w, b0_init_b, b0_t1_w, b0_t1_b, b0_t1_g, b0_t1_be, b0_t1_m, b0_t1_v, b0_t2_w, b0_t2_b, b0_t2_g, b0_t2_be, b0_t2_m, b0_t2_v, b1_init_w, b1_init_b, b1_t1_w, b1_t1_b, b1_t1_g, b1_t1_be, b1_t1_m, b1_t1_v, b1_t2_w, b1_t2_b, b1_t2_g, b1_t2_be, b1_t2_m, b1_t2_v, b2_init_w, b2_init_b, b2_t1_w, b2_t1_b, b2_t1_g, b2_t1_be, b2_t1_m, b2_t1_v, b2_t2_w, b2_t2_b, b2_t2_g, b2_t2_be, b2_t2_m, b2_t2_v, b3_init_w, b3_init_b, b3_t1_w, b3_t1_b, b3_t1_g, b3_t1_be, b3_t1_m, b3_t1_v, b3_t2_w, b3_t2_b, b3_t2_g, b3_t2_be, b3_t2_m, b3_t2_v, b4_init_w, b4_init_b, b4_t1_w, b4_t1_b, b4_t1_g, b4_t1_be, b4_t1_m, b4_t1_v, b4_t2_w, b4_t2_b, b4_t2_g, b4_t2_be, b4_t2_m, b4_t2_v, hm_w1, hm_b1, hm_w2, hm_b2, pf_w1, pf_b1, pf_w2, pf_b2):
    N, Cin, H, W = x.shape
    Wp = _geom(W)
    M = H * Wp
    cin_p = (Cin + 127) // 128 * 128
    n_hm, n_pf = hm_w2.shape[0], pf_w2.shape[0]

    tw = jnp.stack([b0_t1_w, b0_t2_w, b1_t1_w, b1_t2_w, b2_t1_w, b2_t2_w,
                    b3_t1_w, b3_t2_w, b4_t1_w, b4_t2_w])
    tb = jnp.stack([b0_t1_b, b0_t2_b, b1_t1_b, b1_t2_b, b2_t1_b, b2_t2_b,
                    b3_t1_b, b3_t2_b, b4_t1_b, b4_t2_b])
    tg = jnp.stack([b0_t1_g, b0_t2_g, b1_t1_g, b1_t2_g, b2_t1_g, b2_t2_g,
                    b3_t1_g, b3_t2_g, b4_t1_g, b4_t2_g])
    tbe = jnp.stack([b0_t1_be, b0_t2_be, b1_t1_be, b1_t2_be, b2_t1_be,
                     b2_t2_be, b3_t1_be, b3_t2_be, b4_t1_be, b4_t2_be])
    tm = jnp.stack([b0_t1_m, b0_t2_m, b1_t1_m, b1_t2_m, b2_t1_m, b2_t2_m,
                    b3_t1_m, b3_t2_m, b4_t1_m, b4_t2_m])
    tv = jnp.stack([b0_t1_v, b0_t2_v, b1_t1_v, b1_t2_v, b2_t1_v, b2_t2_v,
                    b3_t1_v, b3_t2_v, b4_t1_v, b4_t2_v])
    wa, wb, wc, bc = _tap_weights_all(tw, tb, tg, tbe, tm, tv)

    w00 = jnp.pad(_io(b0_init_w), ((0, cin_p - Cin), (0, 0)))
    w0r = jnp.transpose(
        jnp.stack([b1_init_w, b2_init_w, b3_init_w, b4_init_w])[:, :, :, 0, 0],
        (0, 2, 1))
    b0a = jnp.stack([b0_init_b, b1_init_b, b2_init_b, b3_init_b,
                     b4_init_b])[:, None, :]

    wh1 = jnp.concatenate([_io(hm_w1), _io(pf_w1)], axis=1)
    bh1 = jnp.concatenate([hm_b1, pf_b1]).reshape(1, -1)
    wh2 = jnp.zeros((2 * _C, _C), jnp.float32)
    wh2 = wh2.at[:_C, :n_hm].set(_io(hm_w2))
    wh2 = wh2.at[_C:, n_hm:n_hm + n_pf].set(_io(pf_w2))
    bh2 = jnp.zeros((1, _C), jnp.float32)
    bh2 = bh2.at[0, :n_hm].set(hm_b2)
    bh2 = bh2.at[0, n_hm:n_hm + n_pf].set(pf_b2)

    wlist = [w00, w0r, b0a, wa, wb, wc, bc, wh1, bh1, wh2, bh2]
    wspecs = [pl.BlockSpec(a.shape, lambda b, nd=a.ndim: (0,) * nd)
              for a in wlist]

    xp = jnp.transpose(x, (0, 2, 3, 1)).astype(jnp.float32)
    xp = jnp.pad(xp, ((0, 0), (0, 0), (_G, Wp - W - _G), (0, cin_p - Cin)))
    xp = xp.reshape(N, M, cin_p)
    col = (jnp.arange(M, dtype=jnp.int32) % Wp).reshape(M, 1)

    zb = _ceil8(2 * Wp + 2)
    LD = zb + 2 * Wp + M + 16

    out = pl.pallas_call(
        functools.partial(_stage_kernel, Wp=Wp, W=W, M=M, zb=zb),
        out_shape=jax.ShapeDtypeStruct((N, M, _C), jnp.float32),
        grid=(N,),
        in_specs=[
            pl.BlockSpec((1, M, cin_p), lambda b: (b, 0, 0)),
            pl.BlockSpec((M, 1), lambda b: (0, 0)),
            *wspecs,
        ],
        out_specs=pl.BlockSpec((1, M, _C), lambda b: (b, 0, 0)),
        scratch_shapes=[
            pltpu.VMEM((LD, 3 * _C), jnp.float32),
            pltpu.VMEM((LD, 3 * _C), jnp.float32),
        ],
        compiler_params=pltpu.CompilerParams(
            dimension_semantics=("parallel",)),
    )(xp, col, *wlist)

    outp = out.reshape(N, H, Wp, _C)[:, :, _G:_G + W, :]
    hm = outp[..., :n_hm]
    pf = outp[..., n_hm:n_hm + n_pf]
    return [jnp.transpose(hm, (0, 3, 1, 2)), jnp.transpose(pf, (0, 3, 1, 2))]


# speedup vs baseline: 1.0861x; 1.0861x over previous
import functools

import jax
import jax.numpy as jnp
import numpy as np
from jax.experimental import pallas as pl
from jax.experimental.pallas import tpu as pltpu

_BN_EPS = 1e-5
_C = 128
_G = 4


def _ceil8(v):
    return (v + 7) // 8 * 8


def _geom(W):
    return _ceil8(W + _G + 2)


def _build_shifted(D, t, d, M, Wp, zb):
    dWp = d * Wp
    for kx in range(3):
        b = zb - (kx - 1) * d
        lo, hi = 128 * kx, 128 * (kx + 1)
        above = b - (zb - dWp)
        below = (zb + dWp + M + 8) - (b + M)
        D[pl.ds(zb - dWp, above), lo:hi] = jnp.zeros((above, _C), jnp.float32)
        D[pl.ds(b + M, below), lo:hi] = jnp.zeros((below, _C), jnp.float32)
        D[pl.ds(b, M), lo:hi] = t


def _conv3x3(D, wa_ref, wb_ref, wc_ref, b_ref, *, Wp, d, M, zb):
    dWp = d * Wp
    Mext = M + dWp
    yA = jnp.dot(D[pl.ds(zb - dWp, Mext), 0:256], wa_ref[...],
                 preferred_element_type=jnp.float32)
    yB = jnp.dot(D[pl.ds(zb - dWp, Mext), 256:384], wb_ref[...],
                 preferred_element_type=jnp.float32)
    yC = jnp.dot(D[pl.ds(zb + dWp, M + 8), 0:256], wc_ref[...],
                 preferred_element_type=jnp.float32)
    acc = (jnp.broadcast_to(b_ref[...], (M, _C)).astype(jnp.float32)
           + yA[0:M, 0:128] + yA[dWp:dWp + M, 128:256]
           + yB[0:M, 0:128] + yB[dWp:dWp + M, 128:256]
           + yC[0:M, 0:128] + yC[d:d + M, 128:256])
    return acc


def _stage_kernel(x_ref, col_ref, *refs, Wp, W, M, zb):
    (o_ref,) = refs[-3:-2]
    D1, D2 = refs[-2:]
    wrefs = refs[:-3]
    col = col_ref[...]
    valid = (col >= _G) & (col < _G + W)

    x = x_ref[0]
    for b in range(5):
        (w0, b0, wa1, wb1, wc1, b1,
         wa2, wb2, wc2, b2) = wrefs[10 * b:10 * b + 10]
        init = jnp.dot(x, w0[...], preferred_element_type=jnp.float32)
        init = jnp.where(valid, jnp.maximum(init + b0[...], 0.0), 0.0)
        _build_shifted(D1, init, 1, M, Wp, zb)
        t = _conv3x3(D1, wa1, wb1, wc1, b1, Wp=Wp, d=1, M=M, zb=zb)
        t = jnp.where(valid, jnp.maximum(t, 0.0), 0.0)
        _build_shifted(D2, t, 2, M, Wp, zb)
        t = _conv3x3(D2, wa2, wb2, wc2, b2, Wp=Wp, d=2, M=M, zb=zb)
        t = jnp.where(valid, jnp.maximum(t, 0.0), 0.0)
        x = D1[pl.ds(zb, M), 128:256] + t

    wh1, bh1, wh2, bh2 = wrefs[50:54]
    m = jnp.dot(x, wh1[...], preferred_element_type=jnp.float32)
    m = jnp.maximum(m + bh1[...], 0.0)
    o_ref[0] = jnp.dot(m, wh2[...], preferred_element_type=jnp.float32) + bh2[...]


def _fold_bn(w_oihw, b, g, be, mu, v):
    s = g / jnp.sqrt(v + _BN_EPS)
    return w_oihw * s[:, None, None, None], (b - mu) * s + be


def _io(w_oihw):
    return jnp.transpose(w_oihw[:, :, 0, 0], (1, 0))


def _tap_weights(w_oihw):
    w = jnp.transpose(w_oihw, (2, 3, 1, 0))
    z = jnp.zeros_like(w[2, 2])
    wa = jnp.concatenate(
        [jnp.concatenate([w[0, 0], w[1, 0]], axis=1),
         jnp.concatenate([w[0, 1], w[1, 1]], axis=1)], axis=0)
    wb = jnp.concatenate([w[0, 2], w[1, 2]], axis=1)
    wc = jnp.concatenate(
        [jnp.concatenate([w[2, 0], z], axis=1),
         jnp.concatenate([w[2, 1], w[2, 2]], axis=1)], axis=0)
    return wa, wb, wc


def kernel(x, b0_init_w, b0_init_b, b0_t1_w, b0_t1_b, b0_t1_g, b0_t1_be, b0_t1_m, b0_t1_v, b0_t2_w, b0_t2_b, b0_t2_g, b0_t2_be, b0_t2_m, b0_t2_v, b1_init_w, b1_init_b, b1_t1_w, b1_t1_b, b1_t1_g, b1_t1_be, b1_t1_m, b1_t1_v, b1_t2_w, b1_t2_b, b1_t2_g, b1_t2_be, b1_t2_m, b1_t2_v, b2_init_w, b2_init_b, b2_t1_w, b2_t1_b, b2_t1_g, b2_t1_be, b2_t1_m, b2_t1_v, b2_t2_w, b2_t2_b, b2_t2_g, b2_t2_be, b2_t2_m, b2_t2_v, b3_init_w, b3_init_b, b3_t1_w, b3_t1_b, b3_t1_g, b3_t1_be, b3_t1_m, b3_t1_v, b3_t2_w, b3_t2_b, b3_t2_g, b3_t2_be, b3_t2_m, b3_t2_v, b4_init_w, b4_init_b, b4_t1_w, b4_t1_b, b4_t1_g, b4_t1_be, b4_t1_m, b4_t1_v, b4_t2_w, b4_t2_b, b4_t2_g, b4_t2_be, b4_t2_m, b4_t2_v, hm_w1, hm_b1, hm_w2, hm_b2, pf_w1, pf_b1, pf_w2, pf_b2):
    N, Cin, H, W = x.shape
    Wp = _geom(W)
    M = H * Wp
    cin_p = (Cin + 127) // 128 * 128
    n_hm, n_pf = hm_w2.shape[0], pf_w2.shape[0]

    blocks_raw = [
        (b0_init_w, b0_init_b, b0_t1_w, b0_t1_b, (b0_t1_g, b0_t1_be, b0_t1_m, b0_t1_v),
         b0_t2_w, b0_t2_b, (b0_t2_g, b0_t2_be, b0_t2_m, b0_t2_v)),
        (b1_init_w, b1_init_b, b1_t1_w, b1_t1_b, (b1_t1_g, b1_t1_be, b1_t1_m, b1_t1_v),
         b1_t2_w, b1_t2_b, (b1_t2_g, b1_t2_be, b1_t2_m, b1_t2_v)),
        (b2_init_w, b2_init_b, b2_t1_w, b2_t1_b, (b2_t1_g, b2_t1_be, b2_t1_m, b2_t1_v),
         b2_t2_w, b2_t2_b, (b2_t2_g, b2_t2_be, b2_t2_m, b2_t2_v)),
        (b3_init_w, b3_init_b, b3_t1_w, b3_t1_b, (b3_t1_g, b3_t1_be, b3_t1_m, b3_t1_v),
         b3_t2_w, b3_t2_b, (b3_t2_g, b3_t2_be, b3_t2_m, b3_t2_v)),
        (b4_init_w, b4_init_b, b4_t1_w, b4_t1_b, (b4_t1_g, b4_t1_be, b4_t1_m, b4_t1_v),
         b4_t2_w, b4_t2_b, (b4_t2_g, b4_t2_be, b4_t2_m, b4_t2_v)),
    ]

    wlist, wspecs = [], []

    def add_w(a):
        wlist.append(a)
        wspecs.append(
            pl.BlockSpec(a.shape, lambda b, nd=a.ndim: (0,) * nd))

    for i, (iw, ib, t1w, t1b, t1bn, t2w, t2b, t2bn) in enumerate(blocks_raw):
        w0 = _io(iw)
        if i == 0:
            w0 = jnp.pad(w0, ((0, cin_p - Cin), (0, 0)))
        t1w, t1b = _fold_bn(t1w, t1b, *t1bn)
        t2w, t2b = _fold_bn(t2w, t2b, *t2bn)
        add_w(w0)
        add_w(ib.reshape(1, -1))
        for a in _tap_weights(t1w):
            add_w(a)
        add_w(t1b.reshape(1, -1))
        for a in _tap_weights(t2w):
            add_w(a)
        add_w(t2b.reshape(1, -1))

    wh1 = jnp.concatenate([_io(hm_w1), _io(pf_w1)], axis=1)
    bh1 = jnp.concatenate([hm_b1, pf_b1]).reshape(1, -1)
    wh2 = jnp.zeros((2 * _C, _C), jnp.float32)
    wh2 = wh2.at[:_C, :n_hm].set(_io(hm_w2))
    wh2 = wh2.at[_C:, n_hm:n_hm + n_pf].set(_io(pf_w2))
    bh2 = jnp.zeros((1, _C), jnp.float32)
    bh2 = bh2.at[0, :n_hm].set(hm_b2)
    bh2 = bh2.at[0, n_hm:n_hm + n_pf].set(pf_b2)
    for a in (wh1, bh1, wh2, bh2):
        add_w(a)

    xp = jnp.transpose(x, (0, 2, 3, 1)).astype(jnp.float32)
    xp = jnp.pad(xp, ((0, 0), (0, 0), (_G, Wp - W - _G), (0, cin_p - Cin)))
    xp = xp.reshape(N, M, cin_p)
    col = (jnp.arange(M, dtype=jnp.int32) % Wp).reshape(M, 1)

    zb = _ceil8(2 * Wp + 2)
    LD = zb + 2 * Wp + M + 16

    out = pl.pallas_call(
        functools.partial(_stage_kernel, Wp=Wp, W=W, M=M, zb=zb),
        out_shape=jax.ShapeDtypeStruct((N, M, _C), jnp.float32),
        grid=(N,),
        in_specs=[
            pl.BlockSpec((1, M, cin_p), lambda b: (b, 0, 0)),
            pl.BlockSpec((M, 1), lambda b: (0, 0)),
            *wspecs,
        ],
        out_specs=pl.BlockSpec((1, M, _C), lambda b: (b, 0, 0)),
        scratch_shapes=[
            pltpu.VMEM((LD, 3 * _C), jnp.float32),
            pltpu.VMEM((LD, 3 * _C), jnp.float32),
        ],
        compiler_params=pltpu.CompilerParams(
            dimension_semantics=("parallel",)),
    )(xp, col, *wlist)

    outp = out.reshape(N, H, Wp, _C)[:, :, _G:_G + W, :]
    hm = outp[..., :n_hm]
    pf = outp[..., n_hm:n_hm + n_pf]
    return [jnp.transpose(hm, (0, 3, 1, 2)), jnp.transpose(pf, (0, 3, 1, 2))]


# speedup vs baseline: 1.0870x; 1.0009x over previous
import functools

import jax
import jax.numpy as jnp
import numpy as np
from jax.experimental import pallas as pl
from jax.experimental.pallas import tpu as pltpu

_BN_EPS = 1e-5
_C = 128
_G = 4


def _ceil8(v):
    return (v + 7) // 8 * 8


def _geom(W):
    return _ceil8(W + _G + 2)


def _build_shifted(D, t, d, M, Wp, zb):
    dWp = d * Wp
    for kx in range(3):
        b = zb - (kx - 1) * d
        lo, hi = 128 * kx, 128 * (kx + 1)
        above = b - (zb - dWp)
        below = (zb + dWp + M + 8) - (b + M)
        D[pl.ds(zb - dWp, above), lo:hi] = jnp.zeros((above, _C), jnp.float32)
        D[pl.ds(b + M, below), lo:hi] = jnp.zeros((below, _C), jnp.float32)
        D[pl.ds(b, M), lo:hi] = t


def _conv3x3(D, wa_ref, wb_ref, wc_ref, b_ref, *, Wp, d, M, zb):
    dWp = d * Wp
    Mext = M + dWp
    yA = jnp.dot(D[pl.ds(zb - dWp, Mext), 0:256], wa_ref[...],
                 preferred_element_type=jnp.float32)
    yB = jnp.dot(D[pl.ds(zb - dWp, Mext), 256:384], wb_ref[...],
                 preferred_element_type=jnp.float32)
    yC = jnp.dot(D[pl.ds(zb + dWp, M + 8), 0:256], wc_ref[...],
                 preferred_element_type=jnp.float32)
    acc = (jnp.broadcast_to(b_ref[...], (M, _C)).astype(jnp.float32)
           + yA[0:M, 0:128] + yA[dWp:dWp + M, 128:256]
           + yB[0:M, 0:128] + yB[dWp:dWp + M, 128:256]
           + yC[0:M, 0:128] + yC[d:d + M, 128:256])
    return acc


def _stage_kernel(x_ref, col_ref, *refs, Wp, W, M, zb):
    (o_ref,) = refs[-3:-2]
    D1, D2 = refs[-2:]
    wrefs = refs[:-3]
    col = col_ref[...]
    valid = (col >= _G) & (col < _G + W)

    x = x_ref[0]
    for b in range(5):
        (w0, b0, wa1, wb1, wc1, b1,
         wa2, wb2, wc2, b2) = wrefs[10 * b:10 * b + 10]
        init = jnp.dot(x, w0[...], preferred_element_type=jnp.float32)
        init = jnp.where(valid, jnp.maximum(init + b0[...], 0.0), 0.0)
        _build_shifted(D1, init, 1, M, Wp, zb)
        t = _conv3x3(D1, wa1, wb1, wc1, b1, Wp=Wp, d=1, M=M, zb=zb)
        t = jnp.where(valid, jnp.maximum(t, 0.0), 0.0)
        _build_shifted(D2, t, 2, M, Wp, zb)
        t = _conv3x3(D2, wa2, wb2, wc2, b2, Wp=Wp, d=2, M=M, zb=zb)
        t = jnp.where(valid, jnp.maximum(t, 0.0), 0.0)
        x = D1[pl.ds(zb, M), 128:256] + t

    wh1, bh1, wh2, bh2 = wrefs[50:54]
    m = jnp.dot(x, wh1[...], preferred_element_type=jnp.float32)
    m = jnp.maximum(m + bh1[...], 0.0)
    o_ref[0] = jnp.dot(m, wh2[...], preferred_element_type=jnp.float32) + bh2[...]


def _fold_bn(w_oihw, b, g, be, mu, v):
    s = g / jnp.sqrt(v + _BN_EPS)
    return w_oihw * s[:, None, None, None], (b - mu) * s + be


def _io(w_oihw):
    return jnp.transpose(w_oihw[:, :, 0, 0], (1, 0))


def _tap_weights(w_oihw):
    w = jnp.transpose(w_oihw, (2, 3, 1, 0))
    z = jnp.zeros_like(w[2, 2])
    wa = jnp.concatenate(
        [jnp.concatenate([w[0, 0], w[1, 0]], axis=1),
         jnp.concatenate([w[0, 1], w[1, 1]], axis=1)], axis=0)
    wb = jnp.concatenate([w[0, 2], w[1, 2]], axis=1)
    wc = jnp.concatenate(
        [jnp.concatenate([w[2, 0], z], axis=1),
         jnp.concatenate([w[2, 1], w[2, 2]], axis=1)], axis=0)
    return wa, wb, wc


def kernel(x, b0_init_w, b0_init_b, b0_t1_w, b0_t1_b, b0_t1_g, b0_t1_be, b0_t1_m, b0_t1_v, b0_t2_w, b0_t2_b, b0_t2_g, b0_t2_be, b0_t2_m, b0_t2_v, b1_init_w, b1_init_b, b1_t1_w, b1_t1_b, b1_t1_g, b1_t1_be, b1_t1_m, b1_t1_v, b1_t2_w, b1_t2_b, b1_t2_g, b1_t2_be, b1_t2_m, b1_t2_v, b2_init_w, b2_init_b, b2_t1_w, b2_t1_b, b2_t1_g, b2_t1_be, b2_t1_m, b2_t1_v, b2_t2_w, b2_t2_b, b2_t2_g, b2_t2_be, b2_t2_m, b2_t2_v, b3_init_w, b3_init_b, b3_t1_w, b3_t1_b, b3_t1_g, b3_t1_be, b3_t1_m, b3_t1_v, b3_t2_w, b3_t2_b, b3_t2_g, b3_t2_be, b3_t2_m, b3_t2_v, b4_init_w, b4_init_b, b4_t1_w, b4_t1_b, b4_t1_g, b4_t1_be, b4_t1_m, b4_t1_v, b4_t2_w, b4_t2_b, b4_t2_g, b4_t2_be, b4_t2_m, b4_t2_v, hm_w1, hm_b1, hm_w2, hm_b2, pf_w1, pf_b1, pf_w2, pf_b2):
    N, Cin, H, W = x.shape
    Wp = _geom(W)
    M = H * Wp
    cin_p = (Cin + 63) // 64 * 64
    n_hm, n_pf = hm_w2.shape[0], pf_w2.shape[0]

    blocks_raw = [
        (b0_init_w, b0_init_b, b0_t1_w, b0_t1_b, (b0_t1_g, b0_t1_be, b0_t1_m, b0_t1_v),
         b0_t2_w, b0_t2_b, (b0_t2_g, b0_t2_be, b0_t2_m, b0_t2_v)),
        (b1_init_w, b1_init_b, b1_t1_w, b1_t1_b, (b1_t1_g, b1_t1_be, b1_t1_m, b1_t1_v),
         b1_t2_w, b1_t2_b, (b1_t2_g, b1_t2_be, b1_t2_m, b1_t2_v)),
        (b2_init_w, b2_init_b, b2_t1_w, b2_t1_b, (b2_t1_g, b2_t1_be, b2_t1_m, b2_t1_v),
         b2_t2_w, b2_t2_b, (b2_t2_g, b2_t2_be, b2_t2_m, b2_t2_v)),
        (b3_init_w, b3_init_b, b3_t1_w, b3_t1_b, (b3_t1_g, b3_t1_be, b3_t1_m, b3_t1_v),
         b3_t2_w, b3_t2_b, (b3_t2_g, b3_t2_be, b3_t2_m, b3_t2_v)),
        (b4_init_w, b4_init_b, b4_t1_w, b4_t1_b, (b4_t1_g, b4_t1_be, b4_t1_m, b4_t1_v),
         b4_t2_w, b4_t2_b, (b4_t2_g, b4_t2_be, b4_t2_m, b4_t2_v)),
    ]

    wlist, wspecs = [], []

    def add_w(a):
        wlist.append(a)
        wspecs.append(
            pl.BlockSpec(a.shape, lambda b, nd=a.ndim: (0,) * nd))

    for i, (iw, ib, t1w, t1b, t1bn, t2w, t2b, t2bn) in enumerate(blocks_raw):
        w0 = _io(iw)
        if i == 0:
            w0 = jnp.pad(w0, ((0, cin_p - Cin), (0, 0)))
        t1w, t1b = _fold_bn(t1w, t1b, *t1bn)
        t2w, t2b = _fold_bn(t2w, t2b, *t2bn)
        add_w(w0)
        add_w(ib.reshape(1, -1))
        for a in _tap_weights(t1w):
            add_w(a)
        add_w(t1b.reshape(1, -1))
        for a in _tap_weights(t2w):
            add_w(a)
        add_w(t2b.reshape(1, -1))

    wh1 = jnp.concatenate([_io(hm_w1), _io(pf_w1)], axis=1)
    bh1 = jnp.concatenate([hm_b1, pf_b1]).reshape(1, -1)
    wh2 = jnp.zeros((2 * _C, _C), jnp.float32)
    wh2 = wh2.at[:_C, :n_hm].set(_io(hm_w2))
    wh2 = wh2.at[_C:, n_hm:n_hm + n_pf].set(_io(pf_w2))
    bh2 = jnp.zeros((1, _C), jnp.float32)
    bh2 = bh2.at[0, :n_hm].set(hm_b2)
    bh2 = bh2.at[0, n_hm:n_hm + n_pf].set(pf_b2)
    for a in (wh1, bh1, wh2, bh2):
        add_w(a)

    xp = jnp.transpose(x, (0, 2, 3, 1)).astype(jnp.float32)
    xp = jnp.pad(xp, ((0, 0), (0, 0), (_G, Wp - W - _G), (0, cin_p - Cin)))
    xp = xp.reshape(N, M, cin_p)
    col = (jnp.arange(M, dtype=jnp.int32) % Wp).reshape(M, 1)

    zb = _ceil8(2 * Wp + 2)
    LD = zb + 2 * Wp + M + 16

    out = pl.pallas_call(
        functools.partial(_stage_kernel, Wp=Wp, W=W, M=M, zb=zb),
        out_shape=jax.ShapeDtypeStruct((N, M, _C), jnp.float32),
        grid=(N,),
        in_specs=[
            pl.BlockSpec((1, M, cin_p), lambda b: (b, 0, 0)),
            pl.BlockSpec((M, 1), lambda b: (0, 0)),
            *wspecs,
        ],
        out_specs=pl.BlockSpec((1, M, _C), lambda b: (b, 0, 0)),
        scratch_shapes=[
            pltpu.VMEM((LD, 3 * _C), jnp.float32),
            pltpu.VMEM((LD, 3 * _C), jnp.float32),
        ],
        compiler_params=pltpu.CompilerParams(
            dimension_semantics=("parallel",)),
    )(xp, col, *wlist)

    outp = out.reshape(N, H, Wp, _C)[:, :, _G:_G + W, :]
    hm = outp[..., :n_hm]
    pf = outp[..., n_hm:n_hm + n_pf]
    return [jnp.transpose(hm, (0, 3, 1, 2)), jnp.transpose(pf, (0, 3, 1, 2))]
